# baseline (speedup 1.0000x reference)
import functools

import jax
import jax.numpy as jnp
from jax import lax
from jax.experimental import pallas as pl
from jax.experimental.pallas import tpu as pltpu
from jax.experimental.pallas import tpu_sc as plsc

_LANES = 256
_RING = 8
_NBUF = 2
_SB = 8


@functools.lru_cache(maxsize=None)
def _make_scan(V, D, S, P):
    info = plsc.get_sparse_core_info()
    NC, NS, L = info.num_cores, info.num_subcores, info.num_lanes
    NW = NC * NS
    B = S * P
    NB = (V + _LANES - 1) // _LANES
    BPT = (NB + NW - 1) // NW
    assert B % L == 0 and D % L == 0 and P & (P - 1) == 0
    PSH = P.bit_length() - 1
    BSH = _LANES.bit_length() - 1
    TAIL = 128 * ((V - (NB - 1) * _LANES + 127) // 128)
    mesh = plsc.VectorSubcoreMesh(core_axis_name="c", subcore_axis_name="s")

    @functools.partial(
        pl.kernel,
        mesh=mesh,
        out_type=jax.ShapeDtypeStruct((S, P, D), jnp.float32),
        scratch_types=[
            pltpu.VMEM((B + L,), jnp.int32),
            pltpu.VMEM((B + L,), jnp.int32),
            pltpu.VMEM((B + L,), jnp.int32),
            pltpu.VMEM((B + L,), jnp.int32),
            pltpu.VMEM((B + L,), jnp.int32),
            pltpu.VMEM((_NBUF, D, _LANES), jnp.float32),
            pltpu.VMEM((_RING, D), jnp.float32),
            pltpu.SemaphoreType.DMA,
            pltpu.SemaphoreType.DMA,
            pltpu.SemaphoreType.DMA,
            pltpu.SemaphoreType.DMA,
            pltpu.SemaphoreType.DMA,
        ],
        compiler_params=pltpu.CompilerParams(needs_layout_passes=False),
    )
    def scan_kernel(tab_hbm, idx_hbm, out_hbm, ids_v, kid_v, kpos_v, spos_v,
                    bpk_v, blk_v, row_v, bsem0, bsem1, bsem2, bsem3, rsem):
        bsems = (bsem0, bsem1, bsem2, bsem3)
        wid = lax.axis_index("s") * NC + lax.axis_index("c")
        c0 = wid * BPT
        c1 = jnp.minimum(c0 + BPT, NB)
        lane = lax.iota(jnp.int32, L)

        for s in range(S):
            pltpu.sync_copy(idx_hbm.at[s], ids_v.at[pl.ds(s * P, P)])

        lo = c0 * _LANES
        hi = c1 * _LANES

        def filt(g, cnt):
            v = ids_v[pl.ds(g * L, L)]
            m = jnp.logical_and(v >= lo, v < hi)
            s = plsc.cumsum(m.astype(jnp.int32))
            dst = cnt + s - 1
            plsc.store_scatter(kid_v, [dst], v, mask=m)
            plsc.store_scatter(kpos_v, [dst], g * L + lane, mask=m)
            return cnt + plsc.all_reduce_population_count(m)[0]

        cnt = lax.fori_loop(0, B // L, filt, jnp.int32(0))
        kid_v[pl.ds(cnt, L)] = jnp.full((L,), -1, jnp.int32)
        ngr = (cnt + L - 1) // L

        def start_blk(c, buf):
            @pl.when(c < NB - 1)
            def _():
                pltpu.make_async_copy(
                    tab_hbm.at[:, pl.ds(c * _LANES, _LANES)], blk_v.at[buf],
                    bsems[buf],
                ).start()

            @pl.when(c == NB - 1)
            def _():
                pltpu.make_async_copy(
                    tab_hbm.at[:, pl.ds(c * _LANES, TAIL)],
                    blk_v.at[buf, :, pl.ds(0, TAIL)], bsems[buf],
                ).start()

        def wait_blk(c, buf):
            @pl.when(c < NB - 1)
            def _():
                pltpu.make_async_copy(
                    tab_hbm.at[:, pl.ds(0, _LANES)], blk_v.at[buf], bsems[buf]
                ).wait()

            @pl.when(c == NB - 1)
            def _():
                pltpu.make_async_copy(
                    tab_hbm.at[:, pl.ds(0, TAIL)],
                    blk_v.at[buf, :, pl.ds(0, TAIL)], bsems[buf],
                ).wait()

        start_blk(c0, 0)
        for j in range(1, _NBUF):
            @pl.when(c0 + j < c1)
            def _(j=j):
                start_blk(c0 + j, j)

        def process(c, buf, nout, sngr):
            def mat(g, bcnt):
                v = ids_v[pl.ds(g * L, L)]
                pv = spos_v[pl.ds(g * L, L)]
                m = lax.shift_right_logical(v, BSH) == c
                pk = lax.bitwise_or(
                    lax.shift_left(pv, BSH), lax.bitwise_and(v, _LANES - 1)
                )
                s = plsc.cumsum(m.astype(jnp.int32))
                dst = bcnt + s - 1
                plsc.store_scatter(bpk_v, [dst], pk, mask=m)
                return bcnt + plsc.all_reduce_population_count(m)[0]

            bcnt = lax.fori_loop(0, sngr, mat, jnp.int32(0))

            def emit(t, no):
                pk = bpk_v[pl.ds(t, L)][0]
                pos = lax.shift_right_logical(pk, BSH)
                col = jnp.full((L,), lax.bitwise_and(pk, _LANES - 1), jnp.int32)
                bvec = jnp.full((L,), buf, jnp.int32)
                slot = lax.bitwise_and(no, _RING - 1)

                @pl.when(no >= _RING)
                def _():
                    pltpu.make_async_copy(
                        row_v.at[0], out_hbm.at[0, 0], rsem
                    ).wait()

                for db in range(0, D, L):
                    vals = plsc.load_gather(blk_v, [bvec, db + lane, col])
                    row_v[slot, pl.ds(db, L)] = vals
                pltpu.make_async_copy(
                    row_v.at[slot],
                    out_hbm.at[
                        lax.shift_right_logical(pos, PSH),
                        lax.bitwise_and(pos, P - 1),
                    ],
                    rsem,
                ).start()
                return no + 1

            return lax.fori_loop(0, bcnt, emit, nout)

        def sb_body(si, nout):
            sb0 = c0 + _SB * si
            sb_lo = sb0 * _LANES
            sb_hi = (sb0 + _SB) * _LANES

            def sbmat(g, k):
                v = kid_v[pl.ds(g * L, L)]
                pv = kpos_v[pl.ds(g * L, L)]
                m = jnp.logical_and(v >= sb_lo, v < sb_hi)
                s = plsc.cumsum(m.astype(jnp.int32))
                dst = k + s - 1
                plsc.store_scatter(ids_v, [dst], v, mask=m)
                plsc.store_scatter(spos_v, [dst], pv, mask=m)
                return k + plsc.all_reduce_population_count(m)[0]

            scnt = lax.fori_loop(0, ngr, sbmat, jnp.int32(0))
            ids_v[pl.ds(scnt, L)] = jnp.full((L,), -1, jnp.int32)
            sngr = (scnt + L - 1) // L

            def quad_body(qi, nout):
                cq = sb0 + _NBUF * qi

                def step(j, no):
                    c = cq + j
                    wait_blk(c, j)
                    no = process(c, j, no, sngr)

                    @pl.when(c + _NBUF < c1)
                    def _():
                        start_blk(c + _NBUF, j)

                    return no

                nout = step(0, nout)
                for j in range(1, _NBUF):
                    nout = lax.cond(
                        cq + j < c1, functools.partial(step, j),
                        lambda no: no, nout,
                    )
                return nout

            nquads = lax.div(
                jnp.minimum(_SB, c1 - sb0) + _NBUF - 1, jnp.int32(_NBUF)
            )
            return lax.fori_loop(0, nquads, quad_body, nout)

        nsb = lax.div(c1 - c0 + _SB - 1, jnp.int32(_SB))
        nout = lax.fori_loop(0, nsb, sb_body, jnp.int32(0))

        def drain(i, carry):
            @pl.when(i < jnp.minimum(nout, _RING))
            def _():
                pltpu.make_async_copy(row_v.at[0], out_hbm.at[0, 0], rsem).wait()

            return carry

        lax.fori_loop(0, _RING, drain, 0)

    return scan_kernel


@jax.jit
def kernel(input_ids, table):
    S, P = input_ids.shape
    V, D = table.shape
    idx = input_ids.astype(jnp.int32)
    return _make_scan(V, D, S, P)(table.T, idx)


# speedup vs baseline: 1.1495x; 1.1495x over previous
import functools

import jax
import jax.numpy as jnp
from jax import lax
from jax.experimental import pallas as pl
from jax.experimental.pallas import tpu as pltpu
from jax.experimental.pallas import tpu_sc as plsc

_LANES = 128
_RING = 8
_NBUF = 4
_SB = 16


@functools.lru_cache(maxsize=None)
def _make_scan(V, D, S, P):
    info = plsc.get_sparse_core_info()
    NC, NS, L = info.num_cores, info.num_subcores, info.num_lanes
    NW = NC * NS
    B = S * P
    NB = (V + _LANES - 1) // _LANES
    BPT = (NB + NW - 1) // NW
    assert B % L == 0 and D % L == 0 and P & (P - 1) == 0
    PSH = P.bit_length() - 1
    mesh = plsc.VectorSubcoreMesh(core_axis_name="c", subcore_axis_name="s")

    @functools.partial(
        pl.kernel,
        mesh=mesh,
        out_type=jax.ShapeDtypeStruct((S, P, D), jnp.float32),
        scratch_types=[
            pltpu.VMEM((B + L,), jnp.int32),
            pltpu.VMEM((B + L,), jnp.int32),
            pltpu.VMEM((B + L,), jnp.int32),
            pltpu.VMEM((B + L,), jnp.int32),
            pltpu.VMEM((B + L,), jnp.int32),
            pltpu.VMEM((_NBUF, D, _LANES), jnp.float32),
            pltpu.VMEM((_RING, D), jnp.float32),
            pltpu.SemaphoreType.DMA,
            pltpu.SemaphoreType.DMA,
            pltpu.SemaphoreType.DMA,
            pltpu.SemaphoreType.DMA,
            pltpu.SemaphoreType.DMA,
        ],
        compiler_params=pltpu.CompilerParams(needs_layout_passes=False),
    )
    def scan_kernel(tab_hbm, idx_hbm, out_hbm, ids_v, kid_v, kpos_v, spos_v,
                    bpk_v, blk_v, row_v, bsem0, bsem1, bsem2, bsem3, rsem):
        bsems = (bsem0, bsem1, bsem2, bsem3)
        wid = lax.axis_index("s") * NC + lax.axis_index("c")
        c0 = wid * BPT
        c1 = jnp.minimum(c0 + BPT, NB)
        lane = lax.iota(jnp.int32, L)

        for s in range(S):
            pltpu.sync_copy(idx_hbm.at[s], ids_v.at[pl.ds(s * P, P)])

        lo = c0 * _LANES
        hi = c1 * _LANES

        def filt(g, cnt):
            v = ids_v[pl.ds(g * L, L)]
            m = jnp.logical_and(v >= lo, v < hi)
            s = plsc.cumsum(m.astype(jnp.int32))
            dst = cnt + s - 1
            plsc.store_scatter(kid_v, [dst], v, mask=m)
            plsc.store_scatter(kpos_v, [dst], g * L + lane, mask=m)
            return cnt + plsc.all_reduce_population_count(m)[0]

        cnt = lax.fori_loop(0, B // L, filt, jnp.int32(0))
        kid_v[pl.ds(cnt, L)] = jnp.full((L,), -1, jnp.int32)
        ngr = (cnt + L - 1) // L

        def start_blk(c, buf):
            pltpu.make_async_copy(
                tab_hbm.at[:, pl.ds(c * _LANES, _LANES)], blk_v.at[buf],
                bsems[buf],
            ).start()

        def wait_blk(buf):
            pltpu.make_async_copy(
                tab_hbm.at[:, pl.ds(0, _LANES)], blk_v.at[buf], bsems[buf]
            ).wait()

        start_blk(c0, 0)
        for j in range(1, _NBUF):
            @pl.when(c0 + j < c1)
            def _(j=j):
                start_blk(c0 + j, j)

        def process(c, buf, nout, sngr):
            def mat(g, bcnt):
                v = ids_v[pl.ds(g * L, L)]
                pv = spos_v[pl.ds(g * L, L)]
                m = lax.shift_right_logical(v, 7) == c
                pk = lax.bitwise_or(
                    lax.shift_left(pv, 7), lax.bitwise_and(v, _LANES - 1)
                )
                s = plsc.cumsum(m.astype(jnp.int32))
                dst = bcnt + s - 1
                plsc.store_scatter(bpk_v, [dst], pk, mask=m)
                return bcnt + plsc.all_reduce_population_count(m)[0]

            bcnt = lax.fori_loop(0, sngr, mat, jnp.int32(0))
            wait_blk(buf)

            def emit(t, no):
                pk = bpk_v[pl.ds(t, L)][0]
                pos = lax.shift_right_logical(pk, 7)
                col = jnp.full((L,), lax.bitwise_and(pk, _LANES - 1), jnp.int32)
                bvec = jnp.full((L,), buf, jnp.int32)
                slot = lax.bitwise_and(no, _RING - 1)

                @pl.when(no >= _RING)
                def _():
                    pltpu.make_async_copy(
                        row_v.at[0], out_hbm.at[0, 0], rsem
                    ).wait()

                for db in range(0, D, L):
                    vals = plsc.load_gather(blk_v, [bvec, db + lane, col])
                    row_v[slot, pl.ds(db, L)] = vals
                pltpu.make_async_copy(
                    row_v.at[slot],
                    out_hbm.at[
                        lax.shift_right_logical(pos, PSH),
                        lax.bitwise_and(pos, P - 1),
                    ],
                    rsem,
                ).start()
                return no + 1

            return lax.fori_loop(0, bcnt, emit, nout)

        def sb_body(si, nout):
            sb0 = c0 + _SB * si
            sb_lo = sb0 * _LANES
            sb_hi = (sb0 + _SB) * _LANES

            def sbmat(g, k):
                v = kid_v[pl.ds(g * L, L)]
                pv = kpos_v[pl.ds(g * L, L)]
                m = jnp.logical_and(v >= sb_lo, v < sb_hi)
                s = plsc.cumsum(m.astype(jnp.int32))
                dst = k + s - 1
                plsc.store_scatter(ids_v, [dst], v, mask=m)
                plsc.store_scatter(spos_v, [dst], pv, mask=m)
                return k + plsc.all_reduce_population_count(m)[0]

            scnt = lax.fori_loop(0, ngr, sbmat, jnp.int32(0))
            ids_v[pl.ds(scnt, L)] = jnp.full((L,), -1, jnp.int32)
            sngr = (scnt + L - 1) // L

            def quad_body(qi, nout):
                cq = sb0 + _NBUF * qi

                def step(j, no):
                    c = cq + j
                    no = process(c, j, no, sngr)

                    @pl.when(c + _NBUF < c1)
                    def _():
                        start_blk(c + _NBUF, j)

                    return no

                nout = step(0, nout)
                for j in range(1, _NBUF):
                    nout = lax.cond(
                        cq + j < c1, functools.partial(step, j),
                        lambda no: no, nout,
                    )
                return nout

            nquads = lax.div(
                jnp.minimum(_SB, c1 - sb0) + _NBUF - 1, jnp.int32(_NBUF)
            )
            return lax.fori_loop(0, nquads, quad_body, nout)

        nsb = lax.div(c1 - c0 + _SB - 1, jnp.int32(_SB))
        nout = lax.fori_loop(0, nsb, sb_body, jnp.int32(0))

        def drain(i, carry):
            @pl.when(i < jnp.minimum(nout, _RING))
            def _():
                pltpu.make_async_copy(row_v.at[0], out_hbm.at[0, 0], rsem).wait()

            return carry

        lax.fori_loop(0, _RING, drain, 0)

    return scan_kernel


@jax.jit
def kernel(input_ids, table):
    S, P = input_ids.shape
    V, D = table.shape
    idx = input_ids.astype(jnp.int32)
    return _make_scan(V, D, S, P)(table.T, idx)
